# Initial kernel scaffold; baseline (speedup 1.0000x reference)
import jax
import jax.numpy as jnp
from jax.experimental import pallas as pl


def kernel(z, embedding):
    raise NotImplementedError("write your pallas kernel here")


# speedup vs baseline: 3.3406x; 3.3406x over previous
import jax
import jax.numpy as jnp
from jax.experimental import pallas as pl
from jax.experimental.pallas import tpu as pltpu

N_EMBED = 1024
EMBED_DIM = 64
BETA = 0.25
B = 16
TOK_PER_B = 1024
N_TOK = B * TOK_PER_B


def _vq_body(z_ref, e_ref, zq_ref, enc_ref, idx_ref, loss_ref, perp_ref,
             sse_ref, hist_ref):
    b = pl.program_id(0)

    @pl.when(b == 0)
    def _init():
        sse_ref[...] = jnp.zeros_like(sse_ref)
        hist_ref[...] = jnp.zeros_like(hist_ref)

    zb = z_ref[0]
    zt = zb.T
    e = e_ref[...]

    m = jax.lax.dot_general(zt, e, (((1,), (1,)), ((), ())),
                            preferred_element_type=jnp.float32)
    zsq = jnp.sum(zt * zt, axis=1, keepdims=True)
    esq = jnp.sum(e * e, axis=1)
    d = (zsq + esq[None, :]) - 2.0 * m

    col = jax.lax.broadcasted_iota(jnp.int32, (TOK_PER_B, N_EMBED), 1)
    rowmin = jnp.min(d, axis=1, keepdims=True)
    idx = jnp.min(jnp.where(d == rowmin, col, jnp.int32(1 << 30)), axis=1)
    idx = idx.astype(jnp.int32)
    one_hot = (col == idx[:, None]).astype(jnp.float32)
    enc_ref[...] = one_hot
    idx_ref[0, 0, :] = idx

    zq_t = jax.lax.dot_general(one_hot, e, (((1,), (0,)), ((), ())),
                               preferred_element_type=jnp.float32)
    err = zq_t - zt
    zq_ref[0] = (zt + err).T
    sse_ref[...] += jnp.sum(err * err).reshape(1, 1)
    hist_ref[...] += jnp.sum(one_hot, axis=0, keepdims=True)

    @pl.when(b == B - 1)
    def _finish():
        n_tok = jnp.float32(N_TOK)
        loss_ref[...] = (1.0 + BETA) * sse_ref[...] / (n_tok * EMBED_DIM)
        p = hist_ref[...] / n_tok
        perp_ref[...] = jnp.exp(-jnp.sum(p * jnp.log(p + 1e-10))).reshape(1, 1)


def _run(zr, embedding, interpret=False):
    return pl.pallas_call(
        _vq_body,
        grid=(B,),
        in_specs=[
            pl.BlockSpec((1, EMBED_DIM, TOK_PER_B), lambda b: (b, 0, 0)),
            pl.BlockSpec((N_EMBED, EMBED_DIM), lambda b: (0, 0)),
        ],
        out_specs=[
            pl.BlockSpec((1, EMBED_DIM, TOK_PER_B), lambda b: (b, 0, 0)),
            pl.BlockSpec((TOK_PER_B, N_EMBED), lambda b: (b, 0)),
            pl.BlockSpec((1, 1, TOK_PER_B), lambda b: (b, 0, 0)),
            pl.BlockSpec((1, 1), lambda b: (0, 0)),
            pl.BlockSpec((1, 1), lambda b: (0, 0)),
        ],
        out_shape=[
            jax.ShapeDtypeStruct((B, EMBED_DIM, TOK_PER_B), jnp.float32),
            jax.ShapeDtypeStruct((N_TOK, N_EMBED), jnp.float32),
            jax.ShapeDtypeStruct((B, 1, TOK_PER_B), jnp.int32),
            jax.ShapeDtypeStruct((1, 1), jnp.float32),
            jax.ShapeDtypeStruct((1, 1), jnp.float32),
        ],
        scratch_shapes=[
            pltpu.VMEM((1, 1), jnp.float32),
            pltpu.VMEM((1, N_EMBED), jnp.float32),
        ],
        compiler_params=pltpu.CompilerParams(
            vmem_limit_bytes=100 * 1024 * 1024),
        interpret=interpret,
    )(zr, embedding)


def kernel(z, embedding):
    zr = z.reshape(B, EMBED_DIM, TOK_PER_B)
    zq, enc, idx3, loss, perp = _run(zr, embedding)
    z_q = zq.reshape(z.shape)
    indices = idx3.reshape(N_TOK, 1)
    return (z_q, loss[0, 0], perp[0, 0], enc, indices)


# speedup vs baseline: 3.6333x; 1.0876x over previous
import jax
import jax.numpy as jnp
from jax.experimental import pallas as pl
from jax.experimental.pallas import tpu as pltpu

N_EMBED = 1024
EMBED_DIM = 64
BETA = 0.25
B = 16
TOK_PER_B = 1024
N_TOK = B * TOK_PER_B


def _vq_body(z_ref, e_ref, zq_ref, enc_ref, idx_ref, loss_ref, perp_ref,
             sse_ref, hist_ref):
    b = pl.program_id(0)

    @pl.when(b == 0)
    def _init():
        sse_ref[...] = jnp.zeros_like(sse_ref)
        hist_ref[...] = jnp.zeros_like(hist_ref)

    zb = z_ref[0]
    zt = zb.T
    e = e_ref[...]

    m2 = jax.lax.dot_general(zt * (-2.0), e, (((1,), (1,)), ((), ())),
                             preferred_element_type=jnp.float32)
    zsq = jnp.sum(zt * zt, axis=1, keepdims=True)
    esq = jnp.sum(e * e, axis=1)
    d = (zsq + esq[None, :]) + m2

    col = jax.lax.broadcasted_iota(jnp.int32, (TOK_PER_B, N_EMBED), 1)
    rowmin = jnp.min(d, axis=1, keepdims=True)
    idx = jnp.min(jnp.where(d == rowmin, col, jnp.int32(1 << 30)), axis=1)
    idx = idx.astype(jnp.int32)
    one_hot = (col == idx[:, None]).astype(jnp.float32)
    enc_ref[...] = one_hot
    idx_ref[0, 0, :] = idx

    zq_t = jax.lax.dot_general(one_hot, e, (((1,), (0,)), ((), ())),
                               preferred_element_type=jnp.float32)
    err = zq_t - zt
    zq_ref[0] = (zt + err).T
    sse_ref[...] += jnp.sum(err * err).reshape(1, 1)
    ones_row = jnp.ones((1, TOK_PER_B), jnp.float32)
    hist_ref[...] += jax.lax.dot_general(
        ones_row, one_hot, (((1,), (0,)), ((), ())),
        preferred_element_type=jnp.float32)

    @pl.when(b == B - 1)
    def _finish():
        n_tok = jnp.float32(N_TOK)
        loss_ref[...] = (1.0 + BETA) * sse_ref[...] / (n_tok * EMBED_DIM)
        p = hist_ref[...] / n_tok
        perp_ref[...] = jnp.exp(-jnp.sum(p * jnp.log(p + 1e-10))).reshape(1, 1)


def _run(zr, embedding, interpret=False):
    return pl.pallas_call(
        _vq_body,
        grid=(B,),
        in_specs=[
            pl.BlockSpec((1, EMBED_DIM, TOK_PER_B), lambda b: (b, 0, 0)),
            pl.BlockSpec((N_EMBED, EMBED_DIM), lambda b: (0, 0)),
        ],
        out_specs=[
            pl.BlockSpec((1, EMBED_DIM, TOK_PER_B), lambda b: (b, 0, 0)),
            pl.BlockSpec((TOK_PER_B, N_EMBED), lambda b: (b, 0)),
            pl.BlockSpec((1, 1, TOK_PER_B), lambda b: (b, 0, 0)),
            pl.BlockSpec((1, 1), lambda b: (0, 0)),
            pl.BlockSpec((1, 1), lambda b: (0, 0)),
        ],
        out_shape=[
            jax.ShapeDtypeStruct((B, EMBED_DIM, TOK_PER_B), jnp.float32),
            jax.ShapeDtypeStruct((N_TOK, N_EMBED), jnp.float32),
            jax.ShapeDtypeStruct((B, 1, TOK_PER_B), jnp.int32),
            jax.ShapeDtypeStruct((1, 1), jnp.float32),
            jax.ShapeDtypeStruct((1, 1), jnp.float32),
        ],
        scratch_shapes=[
            pltpu.VMEM((1, 1), jnp.float32),
            pltpu.VMEM((1, N_EMBED), jnp.float32),
        ],
        compiler_params=pltpu.CompilerParams(
            vmem_limit_bytes=100 * 1024 * 1024),
        interpret=interpret,
    )(zr, embedding)


def kernel(z, embedding):
    zr = z.reshape(B, EMBED_DIM, TOK_PER_B)
    zq, enc, idx3, loss, perp = _run(zr, embedding)
    z_q = zq.reshape(z.shape)
    indices = idx3.reshape(N_TOK, 1)
    return (z_q, loss[0, 0], perp[0, 0], enc, indices)


# speedup vs baseline: 3.9005x; 1.0735x over previous
import jax
import jax.numpy as jnp
from jax.experimental import pallas as pl
from jax.experimental.pallas import tpu as pltpu

N_EMBED = 1024
EMBED_DIM = 64
BETA = 0.25
B = 16
TOK_PER_B = 1024
N_TOK = B * TOK_PER_B


def _vq_body(z_ref, e_ref, zq_ref, enc_ref, idx_ref, loss_ref, perp_ref,
             sse_ref, hist_ref, esqb_ref, e2_ref):
    b = pl.program_id(0)

    @pl.when(b == 0)
    def _init():
        sse_ref[...] = jnp.zeros_like(sse_ref)
        hist_ref[...] = jnp.zeros_like(hist_ref)
        eb0 = e_ref[...]
        esq = jnp.sum(eb0 * eb0, axis=1, keepdims=True)
        esqb_ref[...] = jnp.broadcast_to(esq, (N_EMBED, TOK_PER_B))
        e2_ref[...] = eb0 * (-2.0)

    zb = z_ref[0]
    e = e_ref[...]

    m2T = jax.lax.dot_general(e2_ref[...], zb, (((1,), (0,)), ((), ())),
                              preferred_element_type=jnp.float32)
    zsqr = jnp.sum(zb * zb, axis=0, keepdims=True)
    d = (esqb_ref[...] + zsqr) + m2T

    row = jax.lax.broadcasted_iota(jnp.int32, (N_EMBED, TOK_PER_B), 0)
    colmin = jnp.min(d, axis=0, keepdims=True)
    idx_row = jnp.min(jnp.where(d == colmin, row, jnp.int32(1 << 30)),
                      axis=0).astype(jnp.int32)
    idx_ref[0, 0, :] = idx_row

    idx_col = idx_row.reshape(1, TOK_PER_B).T
    col = jax.lax.broadcasted_iota(jnp.int32, (TOK_PER_B, N_EMBED), 1)
    one_hot = (col == idx_col).astype(jnp.float32)
    enc_ref[...] = one_hot

    zq_t = jax.lax.dot_general(one_hot, e, (((1,), (0,)), ((), ())),
                               preferred_element_type=jnp.float32)
    errT = zq_t.T - zb
    zq_ref[0] = zb + errT
    sse_ref[...] += jnp.sum(errT * errT).reshape(1, 1)
    ones_row = jnp.ones((1, TOK_PER_B), jnp.float32)
    hist_ref[...] += jax.lax.dot_general(
        ones_row, one_hot, (((1,), (0,)), ((), ())),
        preferred_element_type=jnp.float32)

    @pl.when(b == B - 1)
    def _finish():
        n_tok = jnp.float32(N_TOK)
        loss_ref[...] = (1.0 + BETA) * sse_ref[...] / (n_tok * EMBED_DIM)
        p = hist_ref[...] / n_tok
        perp_ref[...] = jnp.exp(-jnp.sum(p * jnp.log(p + 1e-10))).reshape(1, 1)


def _run(zr, embedding, interpret=False):
    return pl.pallas_call(
        _vq_body,
        grid=(B,),
        in_specs=[
            pl.BlockSpec((1, EMBED_DIM, TOK_PER_B), lambda b: (b, 0, 0)),
            pl.BlockSpec((N_EMBED, EMBED_DIM), lambda b: (0, 0)),
        ],
        out_specs=[
            pl.BlockSpec((1, EMBED_DIM, TOK_PER_B), lambda b: (b, 0, 0)),
            pl.BlockSpec((TOK_PER_B, N_EMBED), lambda b: (b, 0)),
            pl.BlockSpec((1, 1, TOK_PER_B), lambda b: (b, 0, 0)),
            pl.BlockSpec((1, 1), lambda b: (0, 0)),
            pl.BlockSpec((1, 1), lambda b: (0, 0)),
        ],
        out_shape=[
            jax.ShapeDtypeStruct((B, EMBED_DIM, TOK_PER_B), jnp.float32),
            jax.ShapeDtypeStruct((N_TOK, N_EMBED), jnp.float32),
            jax.ShapeDtypeStruct((B, 1, TOK_PER_B), jnp.int32),
            jax.ShapeDtypeStruct((1, 1), jnp.float32),
            jax.ShapeDtypeStruct((1, 1), jnp.float32),
        ],
        scratch_shapes=[
            pltpu.VMEM((1, 1), jnp.float32),
            pltpu.VMEM((1, N_EMBED), jnp.float32),
            pltpu.VMEM((N_EMBED, TOK_PER_B), jnp.float32),
            pltpu.VMEM((N_EMBED, EMBED_DIM), jnp.float32),
        ],
        compiler_params=pltpu.CompilerParams(
            vmem_limit_bytes=100 * 1024 * 1024),
        interpret=interpret,
    )(zr, embedding)


def kernel(z, embedding):
    zr = z.reshape(B, EMBED_DIM, TOK_PER_B)
    zq, enc, idx3, loss, perp = _run(zr, embedding)
    z_q = zq.reshape(z.shape)
    indices = idx3.reshape(N_TOK, 1)
    return (z_q, loss[0, 0], perp[0, 0], enc, indices)


# speedup vs baseline: 4.3589x; 1.1175x over previous
import jax
import jax.numpy as jnp
from jax.experimental import pallas as pl
from jax.experimental.pallas import tpu as pltpu

N_EMBED = 1024
EMBED_DIM = 64
BETA = 0.25
B = 16
TOK_PER_B = 1024
N_TOK = B * TOK_PER_B
BPB = 2
T = BPB * TOK_PER_B
G = B // BPB


def _vq_body(z_ref, e_ref, zq_ref, enc_ref, idx_ref, loss_ref, perp_ref,
             sse_ref, hist_ref):
    g = pl.program_id(0)

    @pl.when(g == 0)
    def _init():
        sse_ref[...] = jnp.zeros_like(sse_ref)
        hist_ref[...] = jnp.zeros_like(hist_ref)

    zb = jnp.concatenate([z_ref[i] for i in range(BPB)], axis=1)
    e = e_ref[...]

    m2T = jax.lax.dot_general(e * (-2.0), zb, (((1,), (0,)), ((), ())),
                              preferred_element_type=jnp.float32)
    esq = jnp.sum(e * e, axis=1, keepdims=True)
    zsqr = jnp.sum(zb * zb, axis=0, keepdims=True)
    d = (jnp.broadcast_to(esq, (N_EMBED, T)) + zsqr) + m2T

    row = jax.lax.broadcasted_iota(jnp.int32, (N_EMBED, T), 0)
    colmin = jnp.min(d, axis=0, keepdims=True)
    idx_row = jnp.min(jnp.where(d == colmin, row, jnp.int32(1 << 30)),
                      axis=0).astype(jnp.int32)
    idx_ref[...] = idx_row.reshape(BPB, 1, TOK_PER_B)

    idx_col = idx_row.reshape(1, T).T
    col = jax.lax.broadcasted_iota(jnp.int32, (T, N_EMBED), 1)
    one_hot = (col == idx_col).astype(jnp.float32)
    enc_ref[...] = one_hot

    zq_t = jax.lax.dot_general(one_hot, e, (((1,), (0,)), ((), ())),
                               preferred_element_type=jnp.float32)
    errT = zq_t.T - zb
    out = zb + errT
    for i in range(BPB):
        zq_ref[i] = out[:, i * TOK_PER_B:(i + 1) * TOK_PER_B]
    sse_ref[...] += jnp.sum(errT * errT).reshape(1, 1)
    ones_row = jnp.ones((1, T), jnp.float32)
    hist_ref[...] += jax.lax.dot_general(
        ones_row, one_hot, (((1,), (0,)), ((), ())),
        preferred_element_type=jnp.float32)

    @pl.when(g == G - 1)
    def _finish():
        n_tok = jnp.float32(N_TOK)
        loss_ref[...] = (1.0 + BETA) * sse_ref[...] / (n_tok * EMBED_DIM)
        p = hist_ref[...] / n_tok
        perp_ref[...] = jnp.exp(-jnp.sum(p * jnp.log(p + 1e-10))).reshape(1, 1)


def _run(zr, embedding, interpret=False):
    return pl.pallas_call(
        _vq_body,
        grid=(G,),
        in_specs=[
            pl.BlockSpec((BPB, EMBED_DIM, TOK_PER_B), lambda g: (g, 0, 0)),
            pl.BlockSpec((N_EMBED, EMBED_DIM), lambda g: (0, 0)),
        ],
        out_specs=[
            pl.BlockSpec((BPB, EMBED_DIM, TOK_PER_B), lambda g: (g, 0, 0)),
            pl.BlockSpec((T, N_EMBED), lambda g: (g, 0)),
            pl.BlockSpec((BPB, 1, TOK_PER_B), lambda g: (g, 0, 0)),
            pl.BlockSpec((1, 1), lambda g: (0, 0)),
            pl.BlockSpec((1, 1), lambda g: (0, 0)),
        ],
        out_shape=[
            jax.ShapeDtypeStruct((B, EMBED_DIM, TOK_PER_B), jnp.float32),
            jax.ShapeDtypeStruct((N_TOK, N_EMBED), jnp.float32),
            jax.ShapeDtypeStruct((B, 1, TOK_PER_B), jnp.int32),
            jax.ShapeDtypeStruct((1, 1), jnp.float32),
            jax.ShapeDtypeStruct((1, 1), jnp.float32),
        ],
        scratch_shapes=[
            pltpu.VMEM((1, 1), jnp.float32),
            pltpu.VMEM((1, N_EMBED), jnp.float32),
        ],
        compiler_params=pltpu.CompilerParams(
            vmem_limit_bytes=100 * 1024 * 1024),
        interpret=interpret,
    )(zr, embedding)


def kernel(z, embedding):
    zr = z.reshape(B, EMBED_DIM, TOK_PER_B)
    zq, enc, idx3, loss, perp = _run(zr, embedding)
    z_q = zq.reshape(z.shape)
    indices = idx3.reshape(N_TOK, 1)
    return (z_q, loss[0, 0], perp[0, 0], enc, indices)


# speedup vs baseline: 4.6291x; 1.0620x over previous
import jax
import jax.numpy as jnp
from jax.experimental import pallas as pl
from jax.experimental.pallas import tpu as pltpu

N_EMBED = 1024
EMBED_DIM = 64
BETA = 0.25
B = 16
TOK_PER_B = 1024
N_TOK = B * TOK_PER_B
BPB = 2
T = BPB * TOK_PER_B
G = B // BPB


def _vq_body(z_ref, e_ref, zq_ref, enc_ref, idx_ref, loss_ref, perp_ref,
             sse_ref, hist_ref):
    g = pl.program_id(0)

    @pl.when(g == 0)
    def _init():
        sse_ref[...] = jnp.zeros_like(sse_ref)
        hist_ref[...] = jnp.zeros_like(hist_ref)

    zb = jnp.concatenate([z_ref[i] for i in range(BPB)], axis=1)
    e = e_ref[...]

    m2T = jax.lax.dot_general(e * (-2.0), zb, (((1,), (0,)), ((), ())),
                              preferred_element_type=jnp.float32)
    esq = jnp.sum(e * e, axis=1, keepdims=True)
    zsqr = jnp.sum(zb * zb, axis=0, keepdims=True)
    d = (jnp.broadcast_to(esq, (N_EMBED, T)) + zsqr) + m2T

    row = jax.lax.broadcasted_iota(jnp.int32, (N_EMBED, T), 0)
    colmin = jnp.min(d, axis=0, keepdims=True)
    idx_row = jnp.min(jnp.where(d == colmin, row, jnp.int32(1 << 30)),
                      axis=0).astype(jnp.int32)
    idx_ref[...] = idx_row.reshape(BPB, 1, TOK_PER_B)

    idx_col = idx_row.reshape(1, T).T
    col = jax.lax.broadcasted_iota(jnp.int32, (T, N_EMBED), 1)
    one_hot = (col == idx_col).astype(jnp.float32)
    enc_ref[...] = one_hot

    zq_t = jax.lax.dot_general(one_hot, e, (((1,), (0,)), ((), ())),
                               preferred_element_type=jnp.float32)
    errT = zq_t.T - zb
    out = zb + errT
    for i in range(BPB):
        zq_ref[i] = out[:, i * TOK_PER_B:(i + 1) * TOK_PER_B]
    sse_ref[...] += jnp.sum(errT * errT).reshape(1, 1)
    hist_ref[...] += jnp.sum(one_hot, axis=0, keepdims=True)

    @pl.when(g == G - 1)
    def _finish():
        n_tok = jnp.float32(N_TOK)
        loss_ref[...] = (1.0 + BETA) * sse_ref[...] / (n_tok * EMBED_DIM)
        p = hist_ref[...] / n_tok
        perp_ref[...] = jnp.exp(-jnp.sum(p * jnp.log(p + 1e-10))).reshape(1, 1)


def _run(zr, embedding, interpret=False):
    return pl.pallas_call(
        _vq_body,
        grid=(G,),
        in_specs=[
            pl.BlockSpec((BPB, EMBED_DIM, TOK_PER_B), lambda g: (g, 0, 0)),
            pl.BlockSpec((N_EMBED, EMBED_DIM), lambda g: (0, 0)),
        ],
        out_specs=[
            pl.BlockSpec((BPB, EMBED_DIM, TOK_PER_B), lambda g: (g, 0, 0)),
            pl.BlockSpec((T, N_EMBED), lambda g: (g, 0)),
            pl.BlockSpec((BPB, 1, TOK_PER_B), lambda g: (g, 0, 0)),
            pl.BlockSpec((1, 1), lambda g: (0, 0)),
            pl.BlockSpec((1, 1), lambda g: (0, 0)),
        ],
        out_shape=[
            jax.ShapeDtypeStruct((B, EMBED_DIM, TOK_PER_B), jnp.float32),
            jax.ShapeDtypeStruct((N_TOK, N_EMBED), jnp.float32),
            jax.ShapeDtypeStruct((B, 1, TOK_PER_B), jnp.int32),
            jax.ShapeDtypeStruct((1, 1), jnp.float32),
            jax.ShapeDtypeStruct((1, 1), jnp.float32),
        ],
        scratch_shapes=[
            pltpu.VMEM((1, 1), jnp.float32),
            pltpu.VMEM((1, N_EMBED), jnp.float32),
        ],
        compiler_params=pltpu.CompilerParams(
            vmem_limit_bytes=100 * 1024 * 1024),
        interpret=interpret,
    )(zr, embedding)


def kernel(z, embedding):
    zr = z.reshape(B, EMBED_DIM, TOK_PER_B)
    zq, enc, idx3, loss, perp = _run(zr, embedding)
    z_q = zq.reshape(z.shape)
    indices = idx3.reshape(N_TOK, 1)
    return (z_q, loss[0, 0], perp[0, 0], enc, indices)


# speedup vs baseline: 4.6555x; 1.0057x over previous
import jax
import jax.numpy as jnp
from jax.experimental import pallas as pl
from jax.experimental.pallas import tpu as pltpu

N_EMBED = 1024
EMBED_DIM = 64
BETA = 0.25
B = 16
TOK_PER_B = 1024
N_TOK = B * TOK_PER_B
BPB = 2
T = BPB * TOK_PER_B
G = B // BPB


def _vq_body(z_ref, e_ref, zq_ref, enc_ref, idx_ref, loss_ref, perp_ref,
             sse_ref, hist_ref, esqb_ref):
    g = pl.program_id(0)

    @pl.when(g == 0)
    def _init():
        sse_ref[...] = jnp.zeros_like(sse_ref)
        hist_ref[...] = jnp.zeros_like(hist_ref)
        e0 = e_ref[...]
        esq0 = jnp.sum(e0 * e0, axis=1, keepdims=True)
        esqb_ref[...] = jnp.broadcast_to(esq0, (N_EMBED, T))

    zb = jnp.concatenate([z_ref[i] for i in range(BPB)], axis=1)
    e = e_ref[...]

    m2T = jax.lax.dot_general(e * (-2.0), zb, (((1,), (0,)), ((), ())),
                              preferred_element_type=jnp.float32)
    zsqr = jnp.sum(zb * zb, axis=0, keepdims=True)
    d = (esqb_ref[...] + zsqr) + m2T

    row = jax.lax.broadcasted_iota(jnp.int32, (N_EMBED, T), 0)
    colmin = jnp.min(d, axis=0, keepdims=True)
    idx_row = jnp.min(jnp.where(d == colmin, row, jnp.int32(1 << 30)),
                      axis=0).astype(jnp.int32)
    idx_ref[...] = idx_row.reshape(BPB, 1, TOK_PER_B)

    idx_col = idx_row.reshape(1, T).T
    col = jax.lax.broadcasted_iota(jnp.int32, (T, N_EMBED), 1)
    one_hot = (col == idx_col).astype(jnp.float32)
    enc_ref[...] = one_hot

    zq_t = jax.lax.dot_general(one_hot, e, (((1,), (0,)), ((), ())),
                               preferred_element_type=jnp.float32)
    errT = zq_t.T - zb
    out = zb + errT
    for i in range(BPB):
        zq_ref[i] = out[:, i * TOK_PER_B:(i + 1) * TOK_PER_B]
    sse_ref[...] += jnp.sum(errT * errT).reshape(1, 1)
    hist_ref[...] += jnp.sum(one_hot, axis=0, keepdims=True)

    @pl.when(g == G - 1)
    def _finish():
        n_tok = jnp.float32(N_TOK)
        loss_ref[...] = (1.0 + BETA) * sse_ref[...] / (n_tok * EMBED_DIM)
        p = hist_ref[...] / n_tok
        perp_ref[...] = jnp.exp(-jnp.sum(p * jnp.log(p + 1e-10))).reshape(1, 1)


def _run(zr, embedding, interpret=False):
    return pl.pallas_call(
        _vq_body,
        grid=(G,),
        in_specs=[
            pl.BlockSpec((BPB, EMBED_DIM, TOK_PER_B), lambda g: (g, 0, 0)),
            pl.BlockSpec((N_EMBED, EMBED_DIM), lambda g: (0, 0)),
        ],
        out_specs=[
            pl.BlockSpec((BPB, EMBED_DIM, TOK_PER_B), lambda g: (g, 0, 0)),
            pl.BlockSpec((T, N_EMBED), lambda g: (g, 0)),
            pl.BlockSpec((BPB, 1, TOK_PER_B), lambda g: (g, 0, 0)),
            pl.BlockSpec((1, 1), lambda g: (0, 0)),
            pl.BlockSpec((1, 1), lambda g: (0, 0)),
        ],
        out_shape=[
            jax.ShapeDtypeStruct((B, EMBED_DIM, TOK_PER_B), jnp.float32),
            jax.ShapeDtypeStruct((N_TOK, N_EMBED), jnp.float32),
            jax.ShapeDtypeStruct((B, 1, TOK_PER_B), jnp.int32),
            jax.ShapeDtypeStruct((1, 1), jnp.float32),
            jax.ShapeDtypeStruct((1, 1), jnp.float32),
        ],
        scratch_shapes=[
            pltpu.VMEM((1, 1), jnp.float32),
            pltpu.VMEM((1, N_EMBED), jnp.float32),
            pltpu.VMEM((N_EMBED, T), jnp.float32),
        ],
        compiler_params=pltpu.CompilerParams(
            vmem_limit_bytes=100 * 1024 * 1024),
        interpret=interpret,
    )(zr, embedding)


def kernel(z, embedding):
    zr = z.reshape(B, EMBED_DIM, TOK_PER_B)
    zq, enc, idx3, loss, perp = _run(zr, embedding)
    z_q = zq.reshape(z.shape)
    indices = idx3.reshape(N_TOK, 1)
    return (z_q, loss[0, 0], perp[0, 0], enc, indices)
